# Initial kernel scaffold; baseline (speedup 1.0000x reference)
import jax
import jax.numpy as jnp
from jax.experimental import pallas as pl


def kernel(indices, core0, core1, core2):
    raise NotImplementedError("write your pallas kernel here")


# speedup vs baseline: 1.1444x; 1.1444x over previous
import functools

import jax
import jax.numpy as jnp
from jax.experimental import pallas as pl

_P = (100, 100, 100)
_Q = (4, 4, 4)
_R1 = 32
_R2 = 32
_BB = 256


def _tt_block_kernel(idx_ref, c0_ref, c1_ref, c2t_ref, out_ref):
    idx = idx_ref[0, 0, :]
    i0 = idx // (_P[1] * _P[2])
    i1 = (idx // _P[2]) % _P[1]
    i2 = idx % _P[2]

    iota = jax.lax.broadcasted_iota(jnp.int32, (_BB, 128), 1)
    oh0 = (i0[:, None] == iota).astype(jnp.float32)
    oh1 = (i1[:, None] == iota).astype(jnp.float32)
    oh2 = (i2[:, None] == iota).astype(jnp.float32)

    a = jnp.dot(oh0, c0_ref[...], preferred_element_type=jnp.float32)
    m = jnp.dot(oh1, c1_ref[...], preferred_element_type=jnp.float32)
    c = jnp.dot(oh2, c2t_ref[...], preferred_element_type=jnp.float32)

    acc = [jnp.zeros((_BB, 128), jnp.float32) for _ in range(4)]
    for r1 in range(_R1):
        m_r1 = m[:, r1 * 128:(r1 + 1) * 128]
        for q0 in range(4):
            acc[q0] = acc[q0] + a[:, q0 * 32 + r1][:, None] * m_r1

    cols = []
    for q0 in range(4):
        for q1 in range(4):
            t = acc[q0][:, q1 * 32:(q1 + 1) * 32]
            for q2 in range(4):
                s = t * c[:, q2 * 32:(q2 + 1) * 32]
                cols.append(jnp.sum(s, axis=1, keepdims=True))
    out_ref[...] = jnp.concatenate(cols, axis=1)


@jax.jit
def kernel(indices, core0, core1, core2):
    B = indices.shape[0]
    idx = indices.astype(jnp.int32)
    c0 = jnp.pad(core0, ((0, 28), (0, 0)))
    c1 = jnp.pad(core1, ((0, 28), (0, 0)))
    c2t = core2.reshape(_P[2], _R2, _Q[2]).transpose(0, 2, 1).reshape(_P[2], 128)
    c2t = jnp.pad(c2t, ((0, 28), (0, 0)))

    grid = B // _BB
    out = pl.pallas_call(
        _tt_block_kernel,
        grid=(grid,),
        in_specs=[
            pl.BlockSpec((1, 1, _BB), lambda i: (i, 0, 0)),
            pl.BlockSpec((128, 128), lambda i: (0, 0)),
            pl.BlockSpec((128, 4096), lambda i: (0, 0)),
            pl.BlockSpec((128, 128), lambda i: (0, 0)),
        ],
        out_specs=pl.BlockSpec((_BB, 64), lambda i: (i, 0)),
        out_shape=jax.ShapeDtypeStruct((B, 64), jnp.float32),
    )(idx.reshape(grid, 1, _BB), c0, c1, c2t)
    return out
